# baseline (speedup 1.0000x reference)
import functools

import jax
import jax.numpy as jnp
from jax import lax
from jax.experimental import pallas as pl
from jax.experimental.pallas import tpu as pltpu
from jax.experimental.pallas import tpu_sc as plsc

G = 64

NC = 2
NS = 16
K = 128
ZR = 128


def _ceil_to(a, m):
    return (a + m - 1) // m * m


def _sc_segsum_call(h, src2d, dst2d, n_acc, gather):
    e_chunks = dst2d.shape[0]
    dw = h.shape[1]
    steps = e_chunks // (NC * NS)
    rows_per_sub = n_acc // NS
    zsteps = rows_per_sub // ZR
    mesh = plsc.VectorSubcoreMesh(core_axis_name="c", subcore_axis_name="s")

    zrows = jnp.zeros((ZR, dw), dtype=jnp.float32)

    NSLOT = 4

    scratch = [
        pltpu.VMEM((NSLOT, K), jnp.int32),
        pltpu.VMEM((steps, K), jnp.int32),
        pltpu.VMEM((2, K, dw), jnp.float32),
        pltpu.VMEM_SHARED((n_acc, dw), jnp.float32),
        pltpu.SemaphoreType.DMA,
        pltpu.SemaphoreType.DMA,
        pltpu.SemaphoreType.DMA,
        pltpu.SemaphoreType.DMA,
        pltpu.SemaphoreType.DMA,
        pltpu.SemaphoreType.DMA,
    ]

    @functools.partial(
        pl.kernel,
        out_type=jax.ShapeDtypeStruct((NC, n_acc, dw), jnp.float32),
        mesh=mesh,
        scratch_types=scratch,
    )
    def seg_kernel(h_hbm, src_hbm, dst_hbm, z_hbm, out_hbm, idx_s, idx_d,
                   rows, acc, sr0, sr1, si0, si1, si2, si3):
        cid = lax.axis_index("c")
        sid = lax.axis_index("s")
        wid = cid * NS + sid
        cbase = wid * steps
        rsems = (sr0, sr1)
        isems = (si0, si1, si2, si3)

        pltpu.async_copy(dst_hbm.at[pl.ds(cbase, steps)], idx_d, sr0)
        pltpu.sync_copy(z_hbm, rows.at[0])
        row0 = sid * rows_per_sub

        @pl.loop(0, zsteps)
        def _(t):
            pltpu.sync_copy(rows.at[0], acc.at[pl.ds(row0 + t * ZR, ZR)])

        pltpu.make_async_copy(dst_hbm.at[pl.ds(cbase, steps)], idx_d,
                              sr0).wait()

        def start_idx(t, s):
            pltpu.async_copy(src_hbm.at[pl.ds(cbase + t, 1)],
                             idx_s.at[pl.ds(s, 1)], isems[s])

        def wait_idx(s):
            pltpu.make_async_copy(src_hbm.at[pl.ds(cbase, 1)],
                                  idx_s.at[pl.ds(s, 1)], isems[s]).wait()

        def start_gather(t, s, b):
            if gather:
                pltpu.async_copy(h_hbm.at[idx_s.at[s]], rows.at[b],
                                 rsems[b])
            else:
                pltpu.async_copy(h_hbm.at[pl.ds((cbase + t) * K, K)],
                                 rows.at[b], rsems[b])

        def wait_gather(b):
            if gather:
                pltpu.make_async_copy(h_hbm.at[idx_s.at[0]], rows.at[b],
                                      rsems[b]).wait()
            else:
                pltpu.make_async_copy(h_hbm.at[pl.ds(cbase * K, K)],
                                      rows.at[b], rsems[b]).wait()

        if gather:
            for s in range(NSLOT):
                start_idx(s, s)
            wait_idx(0)

        plsc.subcore_barrier()
        start_gather(0, 0, 0)

        @pl.loop(0, steps // NSLOT)
        def _(q):
            t = q * NSLOT
            for c in range(NSLOT):
                tc = t + c
                s_nxt = (c + 1) % NSLOT
                b = c % 2

                @pl.when(tc + 1 < steps)
                def _():
                    if gather:
                        wait_idx(s_nxt)
                    start_gather(tc + 1, s_nxt, 1 - b)

                wait_gather(b)
                pltpu.sync_copy(rows.at[b], acc.at[idx_d.at[tc]], add=True)

                if gather:
                    @pl.when(tc + NSLOT < steps)
                    def _():
                        start_idx(tc + NSLOT, c)

        plsc.subcore_barrier()

        @pl.loop(0, zsteps)
        def _(t):
            pltpu.sync_copy(acc.at[pl.ds(row0 + t * ZR, ZR)],
                            out_hbm.at[cid].at[pl.ds(row0 + t * ZR, ZR)])

    return seg_kernel(h, src2d, dst2d, zrows)


def _layer_body(h_ref, p_ref, e_ref, wt_ref, w_ref, b_ref, o_ref):
    pre = h_ref[...] + p_ref[0] + p_ref[1]
    ea = e_ref[0] + e_ref[1]
    pre = pre + jnp.dot(ea, wt_ref[...], preferred_element_type=jnp.float32)
    z = jnp.dot(pre, w_ref[...], preferred_element_type=jnp.float32)
    o_ref[...] = jnp.maximum(z + b_ref[...], 0.0)


def _tc_layer(h, parts, ea_parts, wtil, w, b2d, blk):
    n, d = h.shape
    de16 = ea_parts.shape[2]
    grid = (n // blk,)
    return pl.pallas_call(
        _layer_body,
        grid=grid,
        in_specs=[
            pl.BlockSpec((blk, d), lambda i: (i, 0)),
            pl.BlockSpec((NC, blk, d), lambda i: (0, i, 0)),
            pl.BlockSpec((NC, blk, de16), lambda i: (0, i, 0)),
            pl.BlockSpec((de16, d), lambda i: (0, 0)),
            pl.BlockSpec((d, d), lambda i: (0, 0)),
            pl.BlockSpec((1, d), lambda i: (0, 0)),
        ],
        out_specs=pl.BlockSpec((blk, d), lambda i: (i, 0)),
        out_shape=jax.ShapeDtypeStruct((n, d), jnp.float32),
    )(h, parts, ea_parts, wtil, w, b2d)


def _pool_body(h_ref, b_ref, fw_ref, fb_ref, o_ref):
    n = h_ref.shape[0]
    seg = lax.broadcasted_iota(jnp.int32, (n, G), 1)
    onehot = (b_ref[...] == seg).astype(jnp.float32)
    sums = lax.dot_general(onehot, h_ref[...], (((0,), (0,)), ((), ())),
                           preferred_element_type=jnp.float32)
    counts = jnp.sum(onehot, axis=0)[:, None]
    pooled = sums / jnp.maximum(counts, 1.0)
    logits = jnp.dot(pooled, fw_ref[...],
                     preferred_element_type=jnp.float32) + fb_ref[...]
    m = jnp.max(logits, axis=1, keepdims=True)
    ex = jnp.exp(logits - m)
    o_ref[...] = ex / jnp.sum(ex, axis=1, keepdims=True)


def _tc_pool(h, batch2d, fc_w, fc_b2d):
    n, d = h.shape
    out = fc_w.shape[1]
    return pl.pallas_call(
        _pool_body,
        in_specs=[
            pl.BlockSpec((n, d), lambda: (0, 0)),
            pl.BlockSpec((n, 1), lambda: (0, 0)),
            pl.BlockSpec((d, out), lambda: (0, 0)),
            pl.BlockSpec((1, out), lambda: (0, 0)),
        ],
        out_specs=pl.BlockSpec((G, out), lambda: (0, 0)),
        out_shape=jax.ShapeDtypeStruct((G, out), jnp.float32),
    )(h, batch2d, fc_w, fc_b2d)


@jax.jit
def kernel(x, edge_index, edge_attr, batch, Ws, bs, Wes, bes, fc_W, fc_b):
    n, d = x.shape
    e = edge_index.shape[1]
    nl, de, _ = Wes.shape

    e_pad = _ceil_to(e, NC * NS * K * 4)
    n_acc = _ceil_to(n, NS * ZR)

    pad = e_pad - e
    src = jnp.concatenate([edge_index[0], jnp.zeros((pad,), jnp.int32)])
    dst = jnp.concatenate([edge_index[1], jnp.full((pad,), n, jnp.int32)])
    src = src.reshape(e_pad // K, K)
    dst = dst.reshape(e_pad // K, K)

    de16 = 16
    ea_aug = jnp.concatenate(
        [edge_attr.astype(jnp.float32),
         jnp.ones((e, 1), jnp.float32),
         jnp.zeros((e, de16 - de - 1), jnp.float32)], axis=1)
    ea_aug = jnp.concatenate([ea_aug, jnp.zeros((pad, de16), jnp.float32)])

    wtil = jnp.concatenate(
        [Wes.astype(jnp.float32),
         bes.astype(jnp.float32)[:, None, :],
         jnp.zeros((nl, de16 - de - 1, d), jnp.float32)], axis=1)

    ea_parts = _sc_segsum_call(ea_aug, src, dst, n_acc, gather=False)

    b2d = bs.astype(jnp.float32)[:, None, :]
    h = x.astype(jnp.float32)
    for l in range(nl):
        parts = _sc_segsum_call(h, src, dst, n_acc, gather=True)
        h = _tc_layer(h, parts[:, :n], ea_parts[:, :n], wtil[l],
                      Ws[l].astype(jnp.float32), b2d[l], blk=2000)

    batch2d = batch.astype(jnp.int32)[:, None]
    return _tc_pool(h, batch2d, fc_W.astype(jnp.float32),
                    fc_b.astype(jnp.float32)[None, :])


# speedup vs baseline: 1.0311x; 1.0311x over previous
import functools

import jax
import jax.numpy as jnp
from jax import lax
from jax.experimental import pallas as pl
from jax.experimental.pallas import tpu as pltpu
from jax.experimental.pallas import tpu_sc as plsc

G = 64

NC = 2
NS = 16
K = 128
ZR = 128


def _ceil_to(a, m):
    return (a + m - 1) // m * m


def _sc_segsum_call(h, src2d, dst2d, n_acc, gather, t0, t1):
    dw = h.shape[1]
    steps0 = t0 // NS
    steps1 = t1 // NS
    max_steps = max(steps0, steps1)
    rows_per_sub = n_acc // NS
    zsteps = rows_per_sub // ZR
    mesh = plsc.VectorSubcoreMesh(core_axis_name="c", subcore_axis_name="s")

    zrows = jnp.zeros((ZR, dw), dtype=jnp.float32)

    NSLOT = 4

    scratch = [
        pltpu.VMEM((NSLOT, K), jnp.int32),
        pltpu.VMEM((max_steps, K), jnp.int32),
        pltpu.VMEM((2, K, dw), jnp.float32),
        pltpu.VMEM_SHARED((n_acc, dw), jnp.float32),
        pltpu.SemaphoreType.DMA,
        pltpu.SemaphoreType.DMA,
        pltpu.SemaphoreType.DMA,
        pltpu.SemaphoreType.DMA,
        pltpu.SemaphoreType.DMA,
        pltpu.SemaphoreType.DMA,
    ]

    @functools.partial(
        pl.kernel,
        out_type=jax.ShapeDtypeStruct((NC, n_acc, dw), jnp.float32),
        mesh=mesh,
        scratch_types=scratch,
    )
    def seg_kernel(h_hbm, src_hbm, dst_hbm, z_hbm, out_hbm, idx_s, idx_d,
                   rows, acc, sr0, sr1, si0, si1, si2, si3):
        cid = lax.axis_index("c")
        sid = lax.axis_index("s")
        steps_c = jnp.where(cid == 0, steps0, steps1)
        cbase = jnp.where(cid == 0, sid * steps0, t0 + sid * steps1)
        rsems = (sr0, sr1)
        isems = (si0, si1, si2, si3)

        pltpu.async_copy(dst_hbm.at[pl.ds(cbase, max_steps)], idx_d, sr0)
        pltpu.sync_copy(z_hbm, rows.at[0])
        row0 = sid * rows_per_sub

        @pl.loop(0, zsteps)
        def _(t):
            pltpu.sync_copy(rows.at[0], acc.at[pl.ds(row0 + t * ZR, ZR)])

        pltpu.make_async_copy(dst_hbm.at[pl.ds(cbase, max_steps)], idx_d,
                              sr0).wait()

        def start_idx(t, s):
            pltpu.async_copy(src_hbm.at[pl.ds(cbase + t, 1)],
                             idx_s.at[pl.ds(s, 1)], isems[s])

        def wait_idx(s):
            pltpu.make_async_copy(src_hbm.at[pl.ds(cbase, 1)],
                                  idx_s.at[pl.ds(s, 1)], isems[s]).wait()

        def start_gather(t, s, b):
            if gather:
                pltpu.async_copy(h_hbm.at[idx_s.at[s]], rows.at[b],
                                 rsems[b])
            else:
                pltpu.async_copy(h_hbm.at[pl.ds((cbase + t) * K, K)],
                                 rows.at[b], rsems[b])

        def wait_gather(b):
            if gather:
                pltpu.make_async_copy(h_hbm.at[idx_s.at[0]], rows.at[b],
                                      rsems[b]).wait()
            else:
                pltpu.make_async_copy(h_hbm.at[pl.ds(cbase * K, K)],
                                      rows.at[b], rsems[b]).wait()

        if gather:
            for s in range(NSLOT):
                start_idx(s, s)
            wait_idx(0)

        plsc.subcore_barrier()
        start_gather(0, 0, 0)

        @pl.loop(0, steps_c // NSLOT)
        def _(q):
            t = q * NSLOT
            for c in range(NSLOT):
                tc = t + c
                s_nxt = (c + 1) % NSLOT
                b = c % 2

                @pl.when(tc + 1 < steps_c)
                def _():
                    if gather:
                        wait_idx(s_nxt)
                    start_gather(tc + 1, s_nxt, 1 - b)

                wait_gather(b)
                pltpu.sync_copy(rows.at[b], acc.at[idx_d.at[tc]], add=True)

                if gather:
                    @pl.when(tc + NSLOT < steps_c)
                    def _():
                        start_idx(tc + NSLOT, c)

        plsc.subcore_barrier()

        @pl.loop(0, zsteps)
        def _(t):
            pltpu.sync_copy(acc.at[pl.ds(row0 + t * ZR, ZR)],
                            out_hbm.at[cid].at[pl.ds(row0 + t * ZR, ZR)])

    return seg_kernel(h, src2d, dst2d, zrows)


def _layer_body(h_ref, p_ref, e_ref, wt_ref, w_ref, b_ref, o_ref):
    pre = h_ref[...] + p_ref[0] + p_ref[1]
    ea = e_ref[0] + e_ref[1]
    pre = pre + jnp.dot(ea, wt_ref[...], preferred_element_type=jnp.float32)
    z = jnp.dot(pre, w_ref[...], preferred_element_type=jnp.float32)
    o_ref[...] = jnp.maximum(z + b_ref[...], 0.0)


def _tc_layer(h, parts, ea_parts, wtil, w, b2d, blk):
    n, d = h.shape
    de16 = ea_parts.shape[2]
    grid = (n // blk,)
    return pl.pallas_call(
        _layer_body,
        grid=grid,
        in_specs=[
            pl.BlockSpec((blk, d), lambda i: (i, 0)),
            pl.BlockSpec((NC, blk, d), lambda i: (0, i, 0)),
            pl.BlockSpec((NC, blk, de16), lambda i: (0, i, 0)),
            pl.BlockSpec((de16, d), lambda i: (0, 0)),
            pl.BlockSpec((d, d), lambda i: (0, 0)),
            pl.BlockSpec((1, d), lambda i: (0, 0)),
        ],
        out_specs=pl.BlockSpec((blk, d), lambda i: (i, 0)),
        out_shape=jax.ShapeDtypeStruct((n, d), jnp.float32),
    )(h, parts, ea_parts, wtil, w, b2d)


def _pool_body(h_ref, b_ref, fw_ref, fb_ref, o_ref):
    n = h_ref.shape[0]
    seg = lax.broadcasted_iota(jnp.int32, (n, G), 1)
    onehot = (b_ref[...] == seg).astype(jnp.float32)
    sums = lax.dot_general(onehot, h_ref[...], (((0,), (0,)), ((), ())),
                           preferred_element_type=jnp.float32)
    counts = jnp.sum(onehot, axis=0)[:, None]
    pooled = sums / jnp.maximum(counts, 1.0)
    logits = jnp.dot(pooled, fw_ref[...],
                     preferred_element_type=jnp.float32) + fb_ref[...]
    m = jnp.max(logits, axis=1, keepdims=True)
    ex = jnp.exp(logits - m)
    o_ref[...] = ex / jnp.sum(ex, axis=1, keepdims=True)


def _tc_pool(h, batch2d, fc_w, fc_b2d):
    n, d = h.shape
    out = fc_w.shape[1]
    return pl.pallas_call(
        _pool_body,
        in_specs=[
            pl.BlockSpec((n, d), lambda: (0, 0)),
            pl.BlockSpec((n, 1), lambda: (0, 0)),
            pl.BlockSpec((d, out), lambda: (0, 0)),
            pl.BlockSpec((1, out), lambda: (0, 0)),
        ],
        out_specs=pl.BlockSpec((G, out), lambda: (0, 0)),
        out_shape=jax.ShapeDtypeStruct((G, out), jnp.float32),
    )(h, batch2d, fc_w, fc_b2d)


@jax.jit
def kernel(x, edge_index, edge_attr, batch, Ws, bs, Wes, bes, fc_W, fc_b):
    n, d = x.shape
    e = edge_index.shape[1]
    nl, de, _ = Wes.shape

    e_pad = _ceil_to(e, NC * NS * K * 4)
    n_acc = _ceil_to(n, NS * ZR)

    U = NS * 4
    tch = e_pad // K
    u = tch // U
    u0 = min(u - 1, max(1, int(round(u * 0.75))))
    t0g, t1g = U * u0, tch - U * u0
    t0e, t1e = U * (u // 2), tch - U * (u // 2)
    pad_ch = max(t0g, t1g, t0e, t1e) // NS

    pad = e_pad - e
    src = jnp.concatenate([edge_index[0], jnp.zeros((pad,), jnp.int32)])
    dst = jnp.concatenate([edge_index[1], jnp.full((pad,), n, jnp.int32)])
    src = jnp.concatenate([src.reshape(tch, K),
                           jnp.zeros((pad_ch, K), jnp.int32)])
    dst = jnp.concatenate([dst.reshape(tch, K),
                           jnp.full((pad_ch, K), n, jnp.int32)])

    de16 = 16
    ea_aug = jnp.concatenate(
        [edge_attr.astype(jnp.float32),
         jnp.ones((e, 1), jnp.float32),
         jnp.zeros((e, de16 - de - 1), jnp.float32)], axis=1)
    ea_aug = jnp.concatenate([ea_aug, jnp.zeros((pad, de16), jnp.float32)])

    wtil = jnp.concatenate(
        [Wes.astype(jnp.float32),
         bes.astype(jnp.float32)[:, None, :],
         jnp.zeros((nl, de16 - de - 1, d), jnp.float32)], axis=1)

    ea_parts = _sc_segsum_call(ea_aug, src, dst, n_acc, gather=False,
                               t0=t0e, t1=t1e)

    b2d = bs.astype(jnp.float32)[:, None, :]
    h = x.astype(jnp.float32)
    for l in range(nl):
        parts = _sc_segsum_call(h, src, dst, n_acc, gather=True,
                                t0=t0g, t1=t1g)
        h = _tc_layer(h, parts[:, :n], ea_parts[:, :n], wtil[l],
                      Ws[l].astype(jnp.float32), b2d[l], blk=2000)

    batch2d = batch.astype(jnp.int32)[:, None]
    return _tc_pool(h, batch2d, fc_W.astype(jnp.float32),
                    fc_b.astype(jnp.float32)[None, :])


# speedup vs baseline: 1.1119x; 1.0783x over previous
import functools

import jax
import jax.numpy as jnp
from jax import lax
from jax.experimental import pallas as pl
from jax.experimental.pallas import tpu as pltpu
from jax.experimental.pallas import tpu_sc as plsc

G = 64

NC = 2
NS = 16
K = 128
ZR = 128


def _ceil_to(a, m):
    return (a + m - 1) // m * m


def _sc_segsum_call(h, src2d, dst2d, n_acc, gather, t0, t1):
    dw = h.shape[1]
    steps0 = t0 // NS
    steps1 = t1 // NS
    rows_per_sub = n_acc // NS
    zsteps = rows_per_sub // ZR
    mesh = plsc.VectorSubcoreMesh(core_axis_name="c", subcore_axis_name="s")

    zrows = jnp.zeros((ZR, dw), dtype=jnp.float32)

    NSLOT = 4

    scratch = [
        pltpu.VMEM((NSLOT, K), jnp.int32),
        pltpu.VMEM((NSLOT, K), jnp.int32),
        pltpu.VMEM((2, K, dw), jnp.float32),
        pltpu.VMEM_SHARED((n_acc, dw), jnp.float32),
        pltpu.SemaphoreType.DMA,
        pltpu.SemaphoreType.DMA,
        pltpu.SemaphoreType.DMA,
        pltpu.SemaphoreType.DMA,
        pltpu.SemaphoreType.DMA,
        pltpu.SemaphoreType.DMA,
    ]

    @functools.partial(
        pl.kernel,
        out_type=jax.ShapeDtypeStruct((NC, n_acc, dw), jnp.float32),
        mesh=mesh,
        scratch_types=scratch,
    )
    def seg_kernel(h_hbm, src_hbm, dst_hbm, z_hbm, out_hbm, idx_s, idx_d,
                   rows, acc, sr0, sr1, si0, si1, si2, si3):
        cid = lax.axis_index("c")
        sid = lax.axis_index("s")
        steps_c = jnp.where(cid == 0, steps0, steps1)
        cbase = jnp.where(cid == 0, sid * steps0, t0 + sid * steps1)
        rsems = (sr0, sr1)
        isems = (si0, si1, si2, si3)

        def start_idx(t, s):
            pltpu.async_copy(dst_hbm.at[pl.ds(cbase + t, 1)],
                             idx_d.at[pl.ds(s, 1)], isems[s])
            if gather:
                pltpu.async_copy(src_hbm.at[pl.ds(cbase + t, 1)],
                                 idx_s.at[pl.ds(s, 1)], isems[s])

        def wait_idx(s):
            pltpu.make_async_copy(dst_hbm.at[pl.ds(cbase, 1)],
                                  idx_d.at[pl.ds(s, 1)], isems[s]).wait()
            if gather:
                pltpu.make_async_copy(src_hbm.at[pl.ds(cbase, 1)],
                                      idx_s.at[pl.ds(s, 1)], isems[s]).wait()

        def start_gather(t, s, b):
            if gather:
                pltpu.async_copy(h_hbm.at[idx_s.at[s]], rows.at[b],
                                 rsems[b])
            else:
                pltpu.async_copy(h_hbm.at[pl.ds((cbase + t) * K, K)],
                                 rows.at[b], rsems[b])

        def wait_gather(b):
            if gather:
                pltpu.make_async_copy(h_hbm.at[idx_s.at[0]], rows.at[b],
                                      rsems[b]).wait()
            else:
                pltpu.make_async_copy(h_hbm.at[pl.ds(cbase * K, K)],
                                      rows.at[b], rsems[b]).wait()

        for s in range(NSLOT):
            start_idx(s, s)

        pltpu.sync_copy(z_hbm, rows.at[0])
        row0 = sid * rows_per_sub

        @pl.loop(0, zsteps)
        def _(t):
            pltpu.sync_copy(rows.at[0], acc.at[pl.ds(row0 + t * ZR, ZR)])

        wait_idx(0)
        plsc.subcore_barrier()
        start_gather(0, 0, 0)

        @pl.loop(0, steps_c // NSLOT)
        def _(q):
            t = q * NSLOT
            for c in range(NSLOT):
                tc = t + c
                s_nxt = (c + 1) % NSLOT
                b = c % 2

                @pl.when(tc + 1 < steps_c)
                def _():
                    wait_idx(s_nxt)
                    start_gather(tc + 1, s_nxt, 1 - b)

                wait_gather(b)
                pltpu.sync_copy(rows.at[b], acc.at[idx_d.at[c]], add=True)

                @pl.when(tc + NSLOT < steps_c)
                def _():
                    start_idx(tc + NSLOT, c)

        plsc.subcore_barrier()

        @pl.loop(0, zsteps)
        def _(t):
            pltpu.sync_copy(acc.at[pl.ds(row0 + t * ZR, ZR)],
                            out_hbm.at[cid].at[pl.ds(row0 + t * ZR, ZR)])

    return seg_kernel(h, src2d, dst2d, zrows)


def _layer_body(h_ref, p_ref, e_ref, wt_ref, w_ref, b_ref, o_ref):
    pre = h_ref[...] + p_ref[0] + p_ref[1]
    ea = e_ref[0] + e_ref[1]
    pre = pre + jnp.dot(ea, wt_ref[...], preferred_element_type=jnp.float32)
    z = jnp.dot(pre, w_ref[...], preferred_element_type=jnp.float32)
    o_ref[...] = jnp.maximum(z + b_ref[...], 0.0)


def _tc_layer(h, parts, ea_parts, wtil, w, b2d, blk):
    n, d = h.shape
    de16 = ea_parts.shape[2]
    grid = (n // blk,)
    return pl.pallas_call(
        _layer_body,
        grid=grid,
        in_specs=[
            pl.BlockSpec((blk, d), lambda i: (i, 0)),
            pl.BlockSpec((NC, blk, d), lambda i: (0, i, 0)),
            pl.BlockSpec((NC, blk, de16), lambda i: (0, i, 0)),
            pl.BlockSpec((de16, d), lambda i: (0, 0)),
            pl.BlockSpec((d, d), lambda i: (0, 0)),
            pl.BlockSpec((1, d), lambda i: (0, 0)),
        ],
        out_specs=pl.BlockSpec((blk, d), lambda i: (i, 0)),
        out_shape=jax.ShapeDtypeStruct((n, d), jnp.float32),
    )(h, parts, ea_parts, wtil, w, b2d)


def _pool_body(h_ref, b_ref, fw_ref, fb_ref, o_ref):
    n = h_ref.shape[0]
    seg = lax.broadcasted_iota(jnp.int32, (n, G), 1)
    onehot = (b_ref[...] == seg).astype(jnp.float32)
    sums = lax.dot_general(onehot, h_ref[...], (((0,), (0,)), ((), ())),
                           preferred_element_type=jnp.float32)
    counts = jnp.sum(onehot, axis=0)[:, None]
    pooled = sums / jnp.maximum(counts, 1.0)
    logits = jnp.dot(pooled, fw_ref[...],
                     preferred_element_type=jnp.float32) + fb_ref[...]
    m = jnp.max(logits, axis=1, keepdims=True)
    ex = jnp.exp(logits - m)
    o_ref[...] = ex / jnp.sum(ex, axis=1, keepdims=True)


def _tc_pool(h, batch2d, fc_w, fc_b2d):
    n, d = h.shape
    out = fc_w.shape[1]
    return pl.pallas_call(
        _pool_body,
        in_specs=[
            pl.BlockSpec((n, d), lambda: (0, 0)),
            pl.BlockSpec((n, 1), lambda: (0, 0)),
            pl.BlockSpec((d, out), lambda: (0, 0)),
            pl.BlockSpec((1, out), lambda: (0, 0)),
        ],
        out_specs=pl.BlockSpec((G, out), lambda: (0, 0)),
        out_shape=jax.ShapeDtypeStruct((G, out), jnp.float32),
    )(h, batch2d, fc_w, fc_b2d)


@jax.jit
def kernel(x, edge_index, edge_attr, batch, Ws, bs, Wes, bes, fc_W, fc_b):
    n, d = x.shape
    e = edge_index.shape[1]
    nl, de, _ = Wes.shape

    e_pad = _ceil_to(e, NC * NS * K * 4)
    n_acc = _ceil_to(n, NS * ZR)

    U = NS * 4
    tch = e_pad // K
    u = tch // U
    u0 = min(u - 1, max(1, int(round(u * 0.975))))
    t0g, t1g = U * u0, tch - U * u0
    t0e, t1e = U * (u // 2), tch - U * (u // 2)
    pad_ch = max(t0g, t1g, t0e, t1e) // NS

    pad = e_pad - e
    src = jnp.concatenate([edge_index[0], jnp.zeros((pad,), jnp.int32)])
    dst = jnp.concatenate([edge_index[1], jnp.full((pad,), n, jnp.int32)])
    src = jnp.concatenate([src.reshape(tch, K),
                           jnp.zeros((pad_ch, K), jnp.int32)])
    dst = jnp.concatenate([dst.reshape(tch, K),
                           jnp.full((pad_ch, K), n, jnp.int32)])

    de16 = 16
    ea_aug = jnp.concatenate(
        [edge_attr.astype(jnp.float32),
         jnp.ones((e, 1), jnp.float32),
         jnp.zeros((e, de16 - de - 1), jnp.float32)], axis=1)
    ea_aug = jnp.concatenate([ea_aug, jnp.zeros((pad, de16), jnp.float32)])

    wtil = jnp.concatenate(
        [Wes.astype(jnp.float32),
         bes.astype(jnp.float32)[:, None, :],
         jnp.zeros((nl, de16 - de - 1, d), jnp.float32)], axis=1)

    ea_parts = _sc_segsum_call(ea_aug, src, dst, n_acc, gather=False,
                               t0=t0e, t1=t1e)

    b2d = bs.astype(jnp.float32)[:, None, :]
    h = x.astype(jnp.float32)
    for l in range(nl):
        parts = _sc_segsum_call(h, src, dst, n_acc, gather=True,
                                t0=t0g, t1=t1g)
        h = _tc_layer(h, parts[:, :n], ea_parts[:, :n], wtil[l],
                      Ws[l].astype(jnp.float32), b2d[l], blk=2000)

    batch2d = batch.astype(jnp.int32)[:, None]
    return _tc_pool(h, batch2d, fc_W.astype(jnp.float32),
                    fc_b.astype(jnp.float32)[None, :])


# speedup vs baseline: 1.1143x; 1.0022x over previous
import functools

import jax
import jax.numpy as jnp
from jax import lax
from jax.experimental import pallas as pl
from jax.experimental.pallas import tpu as pltpu
from jax.experimental.pallas import tpu_sc as plsc

G = 64

NC = 2
NS = 16
K = 128
ZR = 128


def _ceil_to(a, m):
    return (a + m - 1) // m * m


def _sc_segsum_call(h, src2d, dst2d, n_acc, gather, t0, t1):
    dw = h.shape[1]
    steps0 = t0 // NS
    steps1 = t1 // NS
    rows_per_sub = n_acc // NS
    zsteps = rows_per_sub // ZR
    mesh = plsc.VectorSubcoreMesh(core_axis_name="c", subcore_axis_name="s")

    zrows = jnp.zeros((ZR, dw), dtype=jnp.float32)

    NSLOT = 4

    scratch = [
        pltpu.VMEM((NSLOT, K), jnp.int32),
        pltpu.VMEM((NSLOT, K), jnp.int32),
        pltpu.VMEM((2, K, dw), jnp.float32),
        pltpu.VMEM_SHARED((n_acc, dw), jnp.float32),
        pltpu.SemaphoreType.DMA,
        pltpu.SemaphoreType.DMA,
        pltpu.SemaphoreType.DMA,
        pltpu.SemaphoreType.DMA,
        pltpu.SemaphoreType.DMA,
        pltpu.SemaphoreType.DMA,
    ]

    @functools.partial(
        pl.kernel,
        out_type=jax.ShapeDtypeStruct((NC, n_acc, dw), jnp.float32),
        mesh=mesh,
        scratch_types=scratch,
    )
    def seg_kernel(h_hbm, src_hbm, dst_hbm, z_hbm, out_hbm, idx_s, idx_d,
                   rows, acc, sr0, sr1, si0, si1, si2, si3):
        cid = lax.axis_index("c")
        sid = lax.axis_index("s")
        steps_c = jnp.where(cid == 0, steps0, steps1)
        cbase = jnp.where(cid == 0, sid * steps0, t0 + sid * steps1)
        rsems = (sr0, sr1)
        isems = (si0, si1, si2, si3)

        def start_idx(t, s):
            pltpu.async_copy(dst_hbm.at[pl.ds(cbase + t, 1)],
                             idx_d.at[pl.ds(s, 1)], isems[s])
            if gather:
                pltpu.async_copy(src_hbm.at[pl.ds(cbase + t, 1)],
                                 idx_s.at[pl.ds(s, 1)], isems[s])

        def wait_idx(s):
            pltpu.make_async_copy(dst_hbm.at[pl.ds(cbase, 1)],
                                  idx_d.at[pl.ds(s, 1)], isems[s]).wait()
            if gather:
                pltpu.make_async_copy(src_hbm.at[pl.ds(cbase, 1)],
                                      idx_s.at[pl.ds(s, 1)], isems[s]).wait()

        def start_gather(t, s, b):
            if gather:
                pltpu.async_copy(h_hbm.at[idx_s.at[s]], rows.at[b],
                                 rsems[b])
            else:
                pltpu.async_copy(h_hbm.at[pl.ds((cbase + t) * K, K)],
                                 rows.at[b], rsems[b])

        def wait_gather(b):
            if gather:
                pltpu.make_async_copy(h_hbm.at[idx_s.at[0]], rows.at[b],
                                      rsems[b]).wait()
            else:
                pltpu.make_async_copy(h_hbm.at[pl.ds(cbase * K, K)],
                                      rows.at[b], rsems[b]).wait()

        for s in range(NSLOT):
            start_idx(s, s)

        pltpu.sync_copy(z_hbm, rows.at[0])
        row0 = sid * rows_per_sub

        @pl.loop(0, zsteps)
        def _(t):
            pltpu.sync_copy(rows.at[0], acc.at[pl.ds(row0 + t * ZR, ZR)])

        wait_idx(0)
        plsc.subcore_barrier()
        start_gather(0, 0, 0)

        @pl.loop(0, steps_c // NSLOT)
        def _(q):
            t = q * NSLOT
            for c in range(NSLOT):
                tc = t + c
                s_nxt = (c + 1) % NSLOT
                b = c % 2

                @pl.when(tc + 1 < steps_c)
                def _():
                    wait_idx(s_nxt)
                    start_gather(tc + 1, s_nxt, 1 - b)

                wait_gather(b)
                pltpu.sync_copy(rows.at[b], acc.at[idx_d.at[c]], add=True)

                @pl.when(tc + NSLOT < steps_c)
                def _():
                    start_idx(tc + NSLOT, c)

        plsc.subcore_barrier()

        @pl.loop(0, zsteps)
        def _(t):
            pltpu.sync_copy(acc.at[pl.ds(row0 + t * ZR, ZR)],
                            out_hbm.at[cid].at[pl.ds(row0 + t * ZR, ZR)])

    return seg_kernel(h, src2d, dst2d, zrows)


def _layer_body(h_ref, p_ref, e_ref, wt_ref, w_ref, b_ref, o_ref):
    pre = h_ref[...] + p_ref[0] + p_ref[1]
    ea = e_ref[0] + e_ref[1]
    pre = pre + jnp.dot(ea, wt_ref[...], preferred_element_type=jnp.float32)
    z = jnp.dot(pre, w_ref[...], preferred_element_type=jnp.float32)
    o_ref[...] = jnp.maximum(z + b_ref[...], 0.0)


def _tc_layer(h, parts, ea_parts, wtil, w, b2d, blk):
    n, d = h.shape
    de16 = ea_parts.shape[2]
    grid = (n // blk,)
    return pl.pallas_call(
        _layer_body,
        grid=grid,
        in_specs=[
            pl.BlockSpec((blk, d), lambda i: (i, 0)),
            pl.BlockSpec((NC, blk, d), lambda i: (0, i, 0)),
            pl.BlockSpec((NC, blk, de16), lambda i: (0, i, 0)),
            pl.BlockSpec((de16, d), lambda i: (0, 0)),
            pl.BlockSpec((d, d), lambda i: (0, 0)),
            pl.BlockSpec((1, d), lambda i: (0, 0)),
        ],
        out_specs=pl.BlockSpec((blk, d), lambda i: (i, 0)),
        out_shape=jax.ShapeDtypeStruct((n, d), jnp.float32),
    )(h, parts, ea_parts, wtil, w, b2d)


def _pool_body(h_ref, b_ref, fw_ref, fb_ref, o_ref):
    n = h_ref.shape[0]
    seg = lax.broadcasted_iota(jnp.int32, (n, G), 1)
    onehot = (b_ref[...] == seg).astype(jnp.float32)
    sums = lax.dot_general(onehot, h_ref[...], (((0,), (0,)), ((), ())),
                           preferred_element_type=jnp.float32)
    counts = jnp.sum(onehot, axis=0)[:, None]
    pooled = sums / jnp.maximum(counts, 1.0)
    logits = jnp.dot(pooled, fw_ref[...],
                     preferred_element_type=jnp.float32) + fb_ref[...]
    m = jnp.max(logits, axis=1, keepdims=True)
    ex = jnp.exp(logits - m)
    o_ref[...] = ex / jnp.sum(ex, axis=1, keepdims=True)


def _tc_pool(h, batch2d, fc_w, fc_b2d):
    n, d = h.shape
    out = fc_w.shape[1]
    return pl.pallas_call(
        _pool_body,
        in_specs=[
            pl.BlockSpec((n, d), lambda: (0, 0)),
            pl.BlockSpec((n, 1), lambda: (0, 0)),
            pl.BlockSpec((d, out), lambda: (0, 0)),
            pl.BlockSpec((1, out), lambda: (0, 0)),
        ],
        out_specs=pl.BlockSpec((G, out), lambda: (0, 0)),
        out_shape=jax.ShapeDtypeStruct((G, out), jnp.float32),
    )(h, batch2d, fc_w, fc_b2d)


@jax.jit
def kernel(x, edge_index, edge_attr, batch, Ws, bs, Wes, bes, fc_W, fc_b):
    n, d = x.shape
    e = edge_index.shape[1]
    nl, de, _ = Wes.shape

    e_pad = _ceil_to(e, NC * NS * K * 4)
    n_acc = _ceil_to(n, NS * ZR)

    U = NS * 4
    tch = e_pad // K
    u = tch // U
    u0 = min(u - 1, max(1, int(round(u * 0.9))))
    t0g, t1g = U * u0, tch - U * u0
    t0e, t1e = U * (u // 2), tch - U * (u // 2)
    pad_ch = max(t0g, t1g, t0e, t1e) // NS

    pad = e_pad - e
    src = jnp.concatenate([edge_index[0], jnp.zeros((pad,), jnp.int32)])
    dst = jnp.concatenate([edge_index[1], jnp.full((pad,), n, jnp.int32)])
    src = jnp.concatenate([src.reshape(tch, K),
                           jnp.zeros((pad_ch, K), jnp.int32)])
    dst = jnp.concatenate([dst.reshape(tch, K),
                           jnp.full((pad_ch, K), n, jnp.int32)])

    de16 = 16
    ea_aug = jnp.concatenate(
        [edge_attr.astype(jnp.float32),
         jnp.ones((e, 1), jnp.float32),
         jnp.zeros((e, de16 - de - 1), jnp.float32)], axis=1)
    ea_aug = jnp.concatenate([ea_aug, jnp.zeros((pad, de16), jnp.float32)])

    wtil = jnp.concatenate(
        [Wes.astype(jnp.float32),
         bes.astype(jnp.float32)[:, None, :],
         jnp.zeros((nl, de16 - de - 1, d), jnp.float32)], axis=1)

    ea_parts = _sc_segsum_call(ea_aug, src, dst, n_acc, gather=False,
                               t0=t0e, t1=t1e)

    b2d = bs.astype(jnp.float32)[:, None, :]
    h = x.astype(jnp.float32)
    for l in range(nl):
        parts = _sc_segsum_call(h, src, dst, n_acc, gather=True,
                                t0=t0g, t1=t1g)
        h = _tc_layer(h, parts[:, :n], ea_parts[:, :n], wtil[l],
                      Ws[l].astype(jnp.float32), b2d[l], blk=2000)

    batch2d = batch.astype(jnp.int32)[:, None]
    return _tc_pool(h, batch2d, fc_W.astype(jnp.float32),
                    fc_b.astype(jnp.float32)[None, :])
